# baseline (speedup 1.0000x reference)
import functools

import jax
import jax.numpy as jnp
from jax import lax
from jax.experimental import pallas as pl
from jax.experimental.pallas import tpu as pltpu
from jax.experimental.pallas import tpu_sc as plsc

E = 8
TOPK = 2
D = 768
H = 512
EPS = 1e-5

T = 4096
BLK = 256
NBLK = 40
CAP = NBLK * BLK
NW = 32
TW = T // NW
CT = 32


def _router_body(x_ref, wgt_ref, bg_ref, pos_ref, wts_ref, blk_ref, act_ref,
                 xpk_ref):
    x = x_ref[...]
    ulo = lax.bitcast_convert_type(x[:, :D // 2], jnp.uint32)
    uhi = lax.bitcast_convert_type(x[:, D // 2:], jnp.uint32)
    hi_mask = jnp.uint32(0xFFFF0000)
    half = jnp.uint32(0x8000)
    ulo = (ulo + half) & hi_mask
    uhi = (uhi + half) & hi_mask
    xpk_ref[...] = lax.bitcast_convert_type(
        (ulo >> 16) | uhi, jnp.int32)
    gates = jnp.dot(x, wgt_ref[...], preferred_element_type=jnp.float32)
    gates = gates + bg_ref[...]
    eidx = lax.broadcasted_iota(jnp.int32, gates.shape, 1)
    m1 = jnp.max(gates, axis=1, keepdims=True)
    a1 = jnp.min(jnp.where(gates == m1, eidx, E), axis=1, keepdims=True)
    masked = jnp.where(eidx == a1, -jnp.inf, gates)
    m2 = jnp.max(masked, axis=1, keepdims=True)
    a2 = jnp.min(jnp.where(masked == m2, eidx, E), axis=1, keepdims=True)
    z = jnp.exp(m2 - m1)
    w1 = 1.0 / (1.0 + z)
    w2 = z / (1.0 + z)
    wts_ref[...] = jnp.concatenate([w1, w2], axis=1)

    sel1 = (eidx == a1).astype(jnp.float32)
    sel2 = (eidx == a2).astype(jnp.float32)
    counts = sel1 + sel2
    s = 1
    while s < T:
        shifted = jnp.concatenate(
            [jnp.zeros((s, E), jnp.float32), counts[:T - s, :]], axis=0)
        counts = counts + shifted
        s *= 2
    g = counts[T - 1:T, :]
    padded = jnp.ceil(g * (1.0 / BLK)) * BLK
    r8 = lax.broadcasted_iota(jnp.int32, (E, E), 0)
    c8 = lax.broadcasted_iota(jnp.int32, (E, E), 1)
    ut = (r8 <= c8).astype(jnp.float32)
    base_incl = jnp.dot(padded, ut, preferred_element_type=jnp.float32)
    base_excl = base_incl - padded

    pos1 = jnp.sum(jnp.where(eidx == a1, counts + base_excl, 0.0),
                   axis=1, keepdims=True) - 1.0
    pos2 = jnp.sum(jnp.where(eidx == a2, counts + base_excl, 0.0),
                   axis=1, keepdims=True) - 1.0
    pos_ref[...] = jnp.concatenate([pos1, pos2], axis=1).astype(jnp.int32)

    jb = lax.broadcasted_iota(jnp.int32, (64, 1), 0).astype(jnp.float32) * BLK
    cmp = (base_incl <= jb).astype(jnp.float32)
    bexp = jnp.sum(cmp, axis=1, keepdims=True)
    blk_ref[...] = jnp.minimum(bexp, E - 1).astype(jnp.int32)
    total = jnp.sum(padded, axis=1, keepdims=True)
    act_ref[...] = (jb < total).astype(jnp.int32)


@jax.jit
def _router(x_flat, WgT, bg2):
    return pl.pallas_call(
        _router_body,
        out_shape=[
            jax.ShapeDtypeStruct((T, 2), jnp.int32),
            jax.ShapeDtypeStruct((T, 2), jnp.float32),
            jax.ShapeDtypeStruct((64, 1), jnp.int32),
            jax.ShapeDtypeStruct((64, 1), jnp.int32),
            jax.ShapeDtypeStruct((T, D // 2), jnp.int32),
        ],
    )(x_flat, WgT, bg2)


def _dispatch_body(x_hbm, pos_hbm, wts_hbm, xs_hbm, ws_hbm,
                   xrows, idx0, idx1, w0, w1, w0r, w1r, sem):
    wid = lax.axis_index("s") * 2 + lax.axis_index("c")
    base = wid * TW
    dx = pltpu.async_copy(x_hbm.at[pl.ds(base, TW)], xrows, sem)
    di0 = pltpu.async_copy(pos_hbm.at[0, pl.ds(base, TW)], idx0, sem)
    di1 = pltpu.async_copy(pos_hbm.at[1, pl.ds(base, TW)], idx1, sem)
    dw0 = pltpu.async_copy(wts_hbm.at[0, pl.ds(base, TW)], w0, sem)
    dw1 = pltpu.async_copy(wts_hbm.at[1, pl.ds(base, TW)], w1, sem)
    dx.wait()
    di0.wait()
    di1.wait()
    dw0.wait()
    dw1.wait()
    iota16 = lax.iota(jnp.int32, 16)
    zeros16 = jnp.zeros((16,), jnp.float32)
    for g in range(TW // 16):
        w0g = w0[pl.ds(g * 16, 16)]
        w1g = w1[pl.ds(g * 16, 16)]
        for r in range(16):
            row = g * 16 + r
            w0r[row, pl.ds(0, 16)] = jnp.where(iota16 == r, w0g, 0.0)
            w1r[row, pl.ds(0, 16)] = jnp.where(iota16 == r, w1g, 0.0)
            for b in range(1, 8):
                w0r[row, pl.ds(b * 16, 16)] = zeros16
                w1r[row, pl.ds(b * 16, 16)] = zeros16
    d0 = pltpu.async_copy(xrows, xs_hbm.at[idx0], sem)
    d1 = pltpu.async_copy(xrows, xs_hbm.at[idx1], sem)
    d2 = pltpu.async_copy(w0r, ws_hbm.at[idx0], sem)
    d3 = pltpu.async_copy(w1r, ws_hbm.at[idx1], sem)
    d0.wait()
    d1.wait()
    d2.wait()
    d3.wait()


@functools.lru_cache(maxsize=None)
def _make_dispatch():
    return pl.kernel(
        _dispatch_body,
        out_type=[
            jax.ShapeDtypeStruct((CAP, D // 2), jnp.int32),
            jax.ShapeDtypeStruct((CAP, 128), jnp.float32),
        ],
        mesh=plsc.VectorSubcoreMesh(core_axis_name="c", subcore_axis_name="s"),
        scratch_types=[
            pltpu.VMEM((TW, D // 2), jnp.int32),
            pltpu.VMEM((TW,), jnp.int32),
            pltpu.VMEM((TW,), jnp.int32),
            pltpu.VMEM((TW,), jnp.float32),
            pltpu.VMEM((TW,), jnp.float32),
            pltpu.VMEM((TW, 128), jnp.float32),
            pltpu.VMEM((TW, 128), jnp.float32),
            pltpu.SemaphoreType.DMA,
        ],
    )


def _ffn_body(be_ref, act_ref, xs_ref, ws_ref, w1_ref, b1_ref, w2_ref,
              b2_ref, gamma_ref, beta_ref, ys_ref):
    i = pl.program_id(0)

    @pl.when(act_ref[i] == 1)
    def _():
        v = lax.bitcast_convert_type(xs_ref[...], jnp.uint32)
        f_lo = lax.bitcast_convert_type(v << 16, jnp.float32)
        f_hi = lax.bitcast_convert_type(v & jnp.uint32(0xFFFF0000),
                                        jnp.float32)
        xb = jnp.concatenate([f_lo, f_hi], axis=1)
        h = lax.dot_general(xb.astype(jnp.bfloat16), w1_ref[0],
                            (((1,), (1,)), ((), ())),
                            preferred_element_type=jnp.float32) + b1_ref[0]
        h = 0.5 * h * (1.0 + lax.erf(h * 0.7071067811865475))
        y = lax.dot_general(h.astype(jnp.bfloat16), w2_ref[0],
                            (((1,), (1,)), ((), ())),
                            preferred_element_type=jnp.float32) + b2_ref[0]
        y = y + xb
        mu = jnp.mean(y, axis=1, keepdims=True)
        yc = y - mu
        var = jnp.mean(yc * yc, axis=1, keepdims=True)
        yn = yc * lax.rsqrt(var + EPS) * gamma_ref[0] + beta_ref[0]
        ys_ref[...] = yn * jnp.sum(ws_ref[...], axis=1, keepdims=True)


@jax.jit
def _ffn(be, act, xs, ws2, W1T, b1, W2T, b2, gamma, beta):
    grid_spec = pltpu.PrefetchScalarGridSpec(
        num_scalar_prefetch=2,
        grid=(NBLK,),
        in_specs=[
            pl.BlockSpec((BLK, D // 2), lambda i, be, ac: (i, 0)),
            pl.BlockSpec((BLK, 128), lambda i, be, ac: (i, 0)),
            pl.BlockSpec((1, H, D), lambda i, be, ac: (be[i], 0, 0)),
            pl.BlockSpec((1, 1, H), lambda i, be, ac: (be[i], 0, 0)),
            pl.BlockSpec((1, D, H), lambda i, be, ac: (be[i], 0, 0)),
            pl.BlockSpec((1, 1, D), lambda i, be, ac: (be[i], 0, 0)),
            pl.BlockSpec((1, 1, D), lambda i, be, ac: (be[i], 0, 0)),
            pl.BlockSpec((1, 1, D), lambda i, be, ac: (be[i], 0, 0)),
        ],
        out_specs=pl.BlockSpec((BLK, D), lambda i, be, ac: (i, 0)),
    )
    return pl.pallas_call(
        _ffn_body,
        grid_spec=grid_spec,
        out_shape=jax.ShapeDtypeStruct((CAP, D), jnp.float32),
    )(be, act, xs, ws2, W1T, b1, W2T, b2, gamma, beta)


def _combine_body(ys_hbm, pos_hbm, out_hbm, idx0, idx1, r0, r1, sem):
    wid = lax.axis_index("s") * 2 + lax.axis_index("c")
    base = wid * TW
    for c in range(TW // CT):
        pltpu.sync_copy(pos_hbm.at[0, pl.ds(base + c * CT, CT)], idx0)
        pltpu.sync_copy(pos_hbm.at[1, pl.ds(base + c * CT, CT)], idx1)
        d0 = pltpu.async_copy(ys_hbm.at[idx0], r0, sem)
        d1 = pltpu.async_copy(ys_hbm.at[idx1], r1, sem)
        d0.wait()
        d1.wait()

        def _add_row(i, carry):
            for j in range(D // 16):
                sl = pl.ds(j * 16, 16)
                r0[i, sl] = r0[i, sl] + r1[i, sl]
            return carry

        lax.fori_loop(0, CT, _add_row, 0)
        pltpu.sync_copy(r0, out_hbm.at[pl.ds(base + c * CT, CT)])


@functools.lru_cache(maxsize=None)
def _make_combine():
    return pl.kernel(
        _combine_body,
        out_type=jax.ShapeDtypeStruct((T, D), jnp.float32),
        mesh=plsc.VectorSubcoreMesh(core_axis_name="c", subcore_axis_name="s"),
        scratch_types=[
            pltpu.VMEM((CT,), jnp.int32),
            pltpu.VMEM((CT,), jnp.int32),
            pltpu.VMEM((CT, D), jnp.float32),
            pltpu.VMEM((CT, D), jnp.float32),
            pltpu.SemaphoreType.DMA,
        ],
    )


def kernel(x, Wg, bg, W1, b1, W2, b2, gamma, beta):
    orig_shape = x.shape
    x_flat = x.reshape(-1, D)
    pos, wts, blk, act, xpk = _router(x_flat, Wg.T, bg.reshape(1, E))
    pos_t = pos.T
    wts_t = wts.T
    xs, ws = _make_dispatch()(xpk, pos_t, wts_t)
    ys = _ffn(
        blk[:NBLK, 0],
        act[:NBLK, 0],
        xs,
        ws,
        W1.astype(jnp.bfloat16),
        b1.reshape(E, 1, H),
        W2.astype(jnp.bfloat16),
        b2.reshape(E, 1, D),
        gamma.reshape(E, 1, D),
        beta.reshape(E, 1, D),
    )
    out = _make_combine()(ys, pos_t)
    return out.reshape(orig_shape[:-1] + (D,))


# speedup vs baseline: 1.0645x; 1.0645x over previous
import functools

import jax
import jax.numpy as jnp
from jax import lax
from jax.experimental import pallas as pl
from jax.experimental.pallas import tpu as pltpu
from jax.experimental.pallas import tpu_sc as plsc

E = 8
TOPK = 2
D = 768
H = 512
EPS = 1e-5

T = 4096
BLK = 256
NBLK = 40
CAP = NBLK * BLK
NW = 32
TW = T // NW
CT = 32


def _router_body(x_ref, wgt_ref, bg_ref, pos_ref, wts_ref, blk_ref, act_ref,
                 xpk_ref):
    x = x_ref[...]
    ulo = lax.bitcast_convert_type(x[:, :D // 2], jnp.uint32)
    uhi = lax.bitcast_convert_type(x[:, D // 2:], jnp.uint32)
    hi_mask = jnp.uint32(0xFFFF0000)
    half = jnp.uint32(0x8000)
    ulo = (ulo + half) & hi_mask
    uhi = (uhi + half) & hi_mask
    xpk_ref[...] = lax.bitcast_convert_type(
        (ulo >> 16) | uhi, jnp.int32)
    gates = jnp.dot(x, wgt_ref[...], preferred_element_type=jnp.float32)
    gates = gates + bg_ref[...]
    eidx = lax.broadcasted_iota(jnp.int32, gates.shape, 1)
    m1 = jnp.max(gates, axis=1, keepdims=True)
    a1 = jnp.min(jnp.where(gates == m1, eidx, E), axis=1, keepdims=True)
    masked = jnp.where(eidx == a1, -jnp.inf, gates)
    m2 = jnp.max(masked, axis=1, keepdims=True)
    a2 = jnp.min(jnp.where(masked == m2, eidx, E), axis=1, keepdims=True)
    z = jnp.exp(m2 - m1)
    w1 = 1.0 / (1.0 + z)
    w2 = z / (1.0 + z)
    wts_ref[...] = jnp.concatenate([w1, w2], axis=1)

    sel1 = (eidx == a1).astype(jnp.float32)
    sel2 = (eidx == a2).astype(jnp.float32)
    counts = sel1 + sel2
    s = 1
    while s < T:
        shifted = jnp.concatenate(
            [jnp.zeros((s, E), jnp.float32), counts[:T - s, :]], axis=0)
        counts = counts + shifted
        s *= 2
    g = counts[T - 1:T, :]
    padded = jnp.ceil(g * (1.0 / BLK)) * BLK
    r8 = lax.broadcasted_iota(jnp.int32, (E, E), 0)
    c8 = lax.broadcasted_iota(jnp.int32, (E, E), 1)
    ut = (r8 <= c8).astype(jnp.float32)
    base_incl = jnp.dot(padded, ut, preferred_element_type=jnp.float32)
    base_excl = base_incl - padded

    pos1 = jnp.sum(jnp.where(eidx == a1, counts + base_excl, 0.0),
                   axis=1, keepdims=True) - 1.0
    pos2 = jnp.sum(jnp.where(eidx == a2, counts + base_excl, 0.0),
                   axis=1, keepdims=True) - 1.0
    pos_ref[...] = jnp.concatenate([pos1, pos2], axis=1).astype(jnp.int32)

    jb = lax.broadcasted_iota(jnp.int32, (64, 1), 0).astype(jnp.float32) * BLK
    cmp = (base_incl <= jb).astype(jnp.float32)
    bexp = jnp.sum(cmp, axis=1, keepdims=True)
    blk_ref[...] = jnp.minimum(bexp, E - 1).astype(jnp.int32)
    total = jnp.sum(padded, axis=1, keepdims=True)
    act_ref[...] = (jb < total).astype(jnp.int32)


@jax.jit
def _router(x_flat, WgT, bg2):
    return pl.pallas_call(
        _router_body,
        out_shape=[
            jax.ShapeDtypeStruct((T, 2), jnp.int32),
            jax.ShapeDtypeStruct((T, 2), jnp.float32),
            jax.ShapeDtypeStruct((64, 1), jnp.int32),
            jax.ShapeDtypeStruct((64, 1), jnp.int32),
            jax.ShapeDtypeStruct((T, D // 2), jnp.int32),
        ],
    )(x_flat, WgT, bg2)


def _dispatch_body(x_hbm, pos_hbm, wts_hbm, xs_hbm, ws_hbm,
                   xrows, idx0, idx1, w0, w1, w0r, w1r, sem):
    wid = lax.axis_index("s") * 2 + lax.axis_index("c")
    base = wid * TW
    dx = pltpu.async_copy(x_hbm.at[pl.ds(base, TW)], xrows, sem)
    di0 = pltpu.async_copy(pos_hbm.at[0, pl.ds(base, TW)], idx0, sem)
    di1 = pltpu.async_copy(pos_hbm.at[1, pl.ds(base, TW)], idx1, sem)
    dw0 = pltpu.async_copy(wts_hbm.at[0, pl.ds(base, TW)], w0, sem)
    dw1 = pltpu.async_copy(wts_hbm.at[1, pl.ds(base, TW)], w1, sem)
    dx.wait()
    di0.wait()
    di1.wait()
    dw0.wait()
    dw1.wait()
    d0 = pltpu.async_copy(xrows, xs_hbm.at[idx0], sem)
    d1 = pltpu.async_copy(xrows, xs_hbm.at[idx1], sem)
    iota16 = lax.iota(jnp.int32, 16)
    zeros16 = jnp.zeros((16,), jnp.float32)
    for g in range(TW // 16):
        w0g = w0[pl.ds(g * 16, 16)]
        w1g = w1[pl.ds(g * 16, 16)]
        for r in range(16):
            row = g * 16 + r
            w0r[row, pl.ds(0, 16)] = jnp.where(iota16 == r, w0g, 0.0)
            w1r[row, pl.ds(0, 16)] = jnp.where(iota16 == r, w1g, 0.0)
            for b in range(1, 8):
                w0r[row, pl.ds(b * 16, 16)] = zeros16
                w1r[row, pl.ds(b * 16, 16)] = zeros16
    d2 = pltpu.async_copy(w0r, ws_hbm.at[idx0], sem)
    d3 = pltpu.async_copy(w1r, ws_hbm.at[idx1], sem)
    d0.wait()
    d1.wait()
    d2.wait()
    d3.wait()


@functools.lru_cache(maxsize=None)
def _make_dispatch():
    return pl.kernel(
        _dispatch_body,
        out_type=[
            jax.ShapeDtypeStruct((CAP, D // 2), jnp.int32),
            jax.ShapeDtypeStruct((CAP, 128), jnp.float32),
        ],
        mesh=plsc.VectorSubcoreMesh(core_axis_name="c", subcore_axis_name="s"),
        scratch_types=[
            pltpu.VMEM((TW, D // 2), jnp.int32),
            pltpu.VMEM((TW,), jnp.int32),
            pltpu.VMEM((TW,), jnp.int32),
            pltpu.VMEM((TW,), jnp.float32),
            pltpu.VMEM((TW,), jnp.float32),
            pltpu.VMEM((TW, 128), jnp.float32),
            pltpu.VMEM((TW, 128), jnp.float32),
            pltpu.SemaphoreType.DMA,
        ],
    )


def _ffn_body(be_ref, act_ref, xs_ref, ws_ref, w1_ref, b1_ref, w2_ref,
              b2_ref, gamma_ref, beta_ref, ys_ref):
    i = pl.program_id(0)

    @pl.when(act_ref[i] == 1)
    def _():
        v = lax.bitcast_convert_type(xs_ref[...], jnp.uint32)
        f_lo = lax.bitcast_convert_type(v << 16, jnp.float32)
        f_hi = lax.bitcast_convert_type(v & jnp.uint32(0xFFFF0000),
                                        jnp.float32)
        xb = jnp.concatenate([f_lo, f_hi], axis=1)
        h = lax.dot_general(xb.astype(jnp.bfloat16), w1_ref[0],
                            (((1,), (1,)), ((), ())),
                            preferred_element_type=jnp.float32) + b1_ref[0]
        h = 0.5 * h * (1.0 + lax.erf(h * 0.7071067811865475))
        y = lax.dot_general(h.astype(jnp.bfloat16), w2_ref[0],
                            (((1,), (1,)), ((), ())),
                            preferred_element_type=jnp.float32) + b2_ref[0]
        y = y + xb
        mu = jnp.mean(y, axis=1, keepdims=True)
        yc = y - mu
        var = jnp.mean(yc * yc, axis=1, keepdims=True)
        yn = yc * lax.rsqrt(var + EPS) * gamma_ref[0] + beta_ref[0]
        ys_ref[...] = yn * jnp.sum(ws_ref[...], axis=1, keepdims=True)


@jax.jit
def _ffn(be, act, xs, ws2, W1T, b1, W2T, b2, gamma, beta):
    grid_spec = pltpu.PrefetchScalarGridSpec(
        num_scalar_prefetch=2,
        grid=(NBLK,),
        in_specs=[
            pl.BlockSpec((BLK, D // 2), lambda i, be, ac: (i, 0)),
            pl.BlockSpec((BLK, 128), lambda i, be, ac: (i, 0)),
            pl.BlockSpec((1, H, D), lambda i, be, ac: (be[i], 0, 0)),
            pl.BlockSpec((1, 1, H), lambda i, be, ac: (be[i], 0, 0)),
            pl.BlockSpec((1, D, H), lambda i, be, ac: (be[i], 0, 0)),
            pl.BlockSpec((1, 1, D), lambda i, be, ac: (be[i], 0, 0)),
            pl.BlockSpec((1, 1, D), lambda i, be, ac: (be[i], 0, 0)),
            pl.BlockSpec((1, 1, D), lambda i, be, ac: (be[i], 0, 0)),
        ],
        out_specs=pl.BlockSpec((BLK, D), lambda i, be, ac: (i, 0)),
    )
    return pl.pallas_call(
        _ffn_body,
        grid_spec=grid_spec,
        out_shape=jax.ShapeDtypeStruct((CAP, D), jnp.float32),
    )(be, act, xs, ws2, W1T, b1, W2T, b2, gamma, beta)


def _combine_body(ys_hbm, pos_hbm, out_hbm, idx0, idx1,
                  r0a, r1a, r0b, r1b, sema, semb, semwa, semwb, semi):
    wid = lax.axis_index("s") * 2 + lax.axis_index("c")
    base = wid * TW
    di0 = pltpu.async_copy(pos_hbm.at[0, pl.ds(base, TW)], idx0, semi)
    di1 = pltpu.async_copy(pos_hbm.at[1, pl.ds(base, TW)], idx1, semi)
    di0.wait()
    di1.wait()
    nchunks = TW // CT
    bufs = [(r0a, r1a, sema, semwa), (r0b, r1b, semb, semwb)]
    gathers = [None] * nchunks
    writebacks = [None] * nchunks
    for c in range(nchunks + 1):
        if c < nchunks:
            r0, r1, sem, _ = bufs[c % 2]
            if c >= 2 and writebacks[c - 2] is not None:
                writebacks[c - 2].wait()
            sl = pl.ds(c * CT, CT)
            gathers[c] = (
                pltpu.async_copy(ys_hbm.at[idx0.at[sl]], r0, sem),
                pltpu.async_copy(ys_hbm.at[idx1.at[sl]], r1, sem),
            )
        if c >= 1:
            cq = c - 1
            r0, r1, _, semw = bufs[cq % 2]
            gathers[cq][0].wait()
            gathers[cq][1].wait()

            def _add_row(i, carry, r0=r0, r1=r1):
                for j in range(D // 16):
                    sl2 = pl.ds(j * 16, 16)
                    r0[i, sl2] = r0[i, sl2] + r1[i, sl2]
                return carry

            lax.fori_loop(0, CT, _add_row, 0)
            writebacks[cq] = pltpu.async_copy(
                r0, out_hbm.at[pl.ds(base + cq * CT, CT)], semw)
    writebacks[nchunks - 2].wait()
    writebacks[nchunks - 1].wait()


@functools.lru_cache(maxsize=None)
def _make_combine():
    return pl.kernel(
        _combine_body,
        out_type=jax.ShapeDtypeStruct((T, D), jnp.float32),
        mesh=plsc.VectorSubcoreMesh(core_axis_name="c", subcore_axis_name="s"),
        scratch_types=[
            pltpu.VMEM((TW,), jnp.int32),
            pltpu.VMEM((TW,), jnp.int32),
            pltpu.VMEM((CT, D), jnp.float32),
            pltpu.VMEM((CT, D), jnp.float32),
            pltpu.VMEM((CT, D), jnp.float32),
            pltpu.VMEM((CT, D), jnp.float32),
            pltpu.SemaphoreType.DMA,
            pltpu.SemaphoreType.DMA,
            pltpu.SemaphoreType.DMA,
            pltpu.SemaphoreType.DMA,
            pltpu.SemaphoreType.DMA,
        ],
    )


def kernel(x, Wg, bg, W1, b1, W2, b2, gamma, beta):
    orig_shape = x.shape
    x_flat = x.reshape(-1, D)
    pos, wts, blk, act, xpk = _router(x_flat, Wg.T, bg.reshape(1, E))
    pos_t = pos.T
    wts_t = wts.T
    xs, ws = _make_dispatch()(xpk, pos_t, wts_t)
    ys = _ffn(
        blk[:NBLK, 0],
        act[:NBLK, 0],
        xs,
        ws,
        W1.astype(jnp.bfloat16),
        b1.reshape(E, 1, H),
        W2.astype(jnp.bfloat16),
        b2.reshape(E, 1, D),
        gamma.reshape(E, 1, D),
        beta.reshape(E, 1, D),
    )
    out = _make_combine()(ys, pos_t)
    return out.reshape(orig_shape[:-1] + (D,))
